# baseline (speedup 1.0000x reference)
import functools
import math

import jax
import jax.numpy as jnp
import numpy as np
from jax import lax
from jax.experimental import pallas as pl
from jax.experimental.pallas import tpu as pltpu
from jax.experimental.pallas import tpu_sc as plsc

_D = 128
_BATCH = 4
_SEQ = 8192
_N = _BATCH * _SEQ
_SCALE = math.sqrt(float(_D))
_CHUNK = 128
_LANES = 16


def _pe_table(max_seq, d_model):
    position = np.arange(max_seq, dtype=np.float32)[:, None]
    div_term = np.exp(
        np.arange(0, d_model, 2, dtype=np.float32) * (-math.log(10000.0) / d_model)
    )
    pe = np.zeros((max_seq, d_model), dtype=np.float32)
    pe[:, 0::2] = np.sin(position * div_term)
    pe[:, 1::2] = np.cos(position * div_term)
    return jnp.asarray(pe)


@functools.cache
def _build_sc_kernel():
    info = plsc.get_sparse_core_info()
    nc, ns = info.num_cores, info.num_subcores
    nw = nc * ns
    per_w = _N // nw
    n_chunks = per_w // _CHUNK

    mesh = plsc.VectorSubcoreMesh(core_axis_name="c", subcore_axis_name="s")

    @functools.partial(
        pl.kernel,
        out_type=jax.ShapeDtypeStruct((_N, _D), jnp.float32),
        mesh=mesh,
        scratch_types=[
            pltpu.VMEM((per_w,), jnp.int32),
            pltpu.VMEM((_CHUNK, _D), jnp.float32),
            pltpu.VMEM((_CHUNK, _D), jnp.float32),
            pltpu.SemaphoreType.DMA,
        ],
    )
    def emb_kernel(x_hbm, pe_hbm, table_hbm, out_hbm, idx_v, rows_v, pe_v, sem):
        wid = lax.axis_index("s") * nc + lax.axis_index("c")
        base = wid * per_w
        pos0 = lax.rem(base, _SEQ)

        pltpu.sync_copy(x_hbm.at[pl.ds(base, per_w)], idx_v)

        def chunk_body(c, carry):
            off = c * _CHUNK
            pltpu.async_copy(
                table_hbm.at[idx_v.at[pl.ds(off, _CHUNK)]], rows_v, sem
            ).wait()
            pltpu.sync_copy(pe_hbm.at[pl.ds(pos0 + off, _CHUNK)], pe_v)

            def row_body(r, carry2):
                for j in range(_D // _LANES):
                    sl = pl.ds(j * _LANES, _LANES)
                    rows_v[r, sl] = rows_v[r, sl] * _SCALE + pe_v[r, sl]
                return carry2

            lax.fori_loop(0, _CHUNK, row_body, 0)
            pltpu.sync_copy(rows_v, out_hbm.at[pl.ds(base + off, _CHUNK)])
            return carry

        lax.fori_loop(0, n_chunks, chunk_body, 0)

    return emb_kernel


def kernel(x, table):
    pe = _pe_table(_SEQ, _D)
    xf = x.reshape(_N)
    out = _build_sc_kernel()(xf, pe, table)
    return out.reshape(_BATCH, _SEQ, _D)


# speedup vs baseline: 1.4624x; 1.4624x over previous
import functools
import math

import jax
import jax.numpy as jnp
import numpy as np
from jax import lax
from jax.experimental import pallas as pl
from jax.experimental.pallas import tpu as pltpu
from jax.experimental.pallas import tpu_sc as plsc

_D = 128
_BATCH = 4
_SEQ = 8192
_N = _BATCH * _SEQ
_SCALE = math.sqrt(float(_D))
_CHUNK = 128
_LANES = 16
_NBUF = 3


def _pe_table(max_seq, d_model):
    position = np.arange(max_seq, dtype=np.float32)[:, None]
    div_term = np.exp(
        np.arange(0, d_model, 2, dtype=np.float32) * (-math.log(10000.0) / d_model)
    )
    pe = np.zeros((max_seq, d_model), dtype=np.float32)
    pe[:, 0::2] = np.sin(position * div_term)
    pe[:, 1::2] = np.cos(position * div_term)
    return jnp.asarray(pe)


@functools.cache
def _build_sc_kernel():
    info = plsc.get_sparse_core_info()
    nc, ns = info.num_cores, info.num_subcores
    nw = nc * ns
    per_w = _N // nw
    pos_per_w = _SEQ // nw
    halves = pos_per_w // _CHUNK
    n_chunks = per_w // _CHUNK

    mesh = plsc.VectorSubcoreMesh(core_axis_name="c", subcore_axis_name="s")

    @functools.partial(
        pl.kernel,
        out_type=jax.ShapeDtypeStruct((_N, _D), jnp.float32),
        mesh=mesh,
        scratch_types=[
            pltpu.VMEM((per_w,), jnp.int32),
            pltpu.VMEM((pos_per_w, _D), jnp.float32),
            pltpu.VMEM((_NBUF, _CHUNK, _D), jnp.float32),
            pltpu.SemaphoreType.DMA,
            pltpu.SemaphoreType.DMA,
            pltpu.SemaphoreType.DMA,
        ],
    )
    def emb_kernel(
        x_hbm, pe_hbm, table_hbm, out_hbm, idx_v, pe_v, rows, sem_g, sem_o, sem_p
    ):
        wid = lax.axis_index("s") * nc + lax.axis_index("c")
        pos0 = wid * pos_per_w

        for b in range(_BATCH):
            pltpu.sync_copy(
                x_hbm.at[pl.ds(b * _SEQ + pos0, pos_per_w)],
                idx_v.at[pl.ds(b * pos_per_w, pos_per_w)],
            )
        pe_cp = pltpu.async_copy(pe_hbm.at[pl.ds(pos0, pos_per_w)], pe_v, sem_p)

        def start_gather(c):
            return pltpu.async_copy(
                table_hbm.at[idx_v.at[pl.ds(c * _CHUNK, _CHUNK)]],
                rows.at[c % _NBUF],
                sem_g,
            )

        g_cp = [None] * n_chunks
        w_cp = [None] * n_chunks
        g_cp[0] = start_gather(0)
        g_cp[1] = start_gather(1)
        pe_cp.wait()

        for c in range(n_chunks):
            g_cp[c].wait()
            if c + 2 < n_chunks:
                if c >= 1:
                    w_cp[c - 1].wait()
                g_cp[c + 2] = start_gather(c + 2)

            b, h = divmod(c, halves)
            buf = c % _NBUF

            def row_body(r, carry, buf=buf, h=h):
                for j in range(_D // _LANES):
                    sl = pl.ds(j * _LANES, _LANES)
                    rows[buf, r, sl] = (
                        rows[buf, r, sl] * _SCALE + pe_v[h * _CHUNK + r, sl]
                    )
                return carry

            lax.fori_loop(0, _CHUNK, row_body, 0)

            out_base = b * _SEQ + pos0 + h * _CHUNK
            w_cp[c] = pltpu.async_copy(
                rows.at[buf], out_hbm.at[pl.ds(out_base, _CHUNK)], sem_o
            )

        for c in range(n_chunks - _NBUF, n_chunks):
            w_cp[c].wait()

    return emb_kernel


def kernel(x, table):
    pe = _pe_table(_SEQ, _D)
    xf = x.reshape(_N)
    out = _build_sc_kernel()(xf, pe, table)
    return out.reshape(_BATCH, _SEQ, _D)
